# Initial kernel scaffold; baseline (speedup 1.0000x reference)
import jax
import jax.numpy as jnp
from jax.experimental import pallas as pl


def kernel(tokens, charges, aa_table, charge_table):
    raise NotImplementedError("write your pallas kernel here")


# speedup vs baseline: 4.6397x; 4.6397x over previous
import math

import jax
import jax.numpy as jnp
from jax import lax
from jax.experimental import pallas as pl
from jax.experimental.pallas import tpu as pltpu

B = 1024
L = 200
D = 512
NV = 65
NC = 11
NT = NV + NC
BLK = 3200


def _body(fidx_ref, tab_ref, out_ref, pe_ref):
    @pl.when(pl.program_id(0) == 0)
    def _():
        d_idx = lax.broadcasted_iota(jnp.int32, (L, D), 1)
        pos = lax.broadcasted_iota(jnp.int32, (L, D), 0).astype(jnp.float32)
        d_even = ((d_idx // 2) * 2).astype(jnp.float32)
        ang = pos * jnp.exp(d_even * (-math.log(10000.0) / D))
        pe = jnp.where(d_idx % 2 == 0, jnp.sin(ang), jnp.cos(ang))
        for k in range(BLK // L):
            pe_ref[k * L:(k + 1) * L, :] = pe

    tok = fidx_ref[:, 0:1]
    cid = fidx_ref[:, 1:2]
    cols = lax.broadcasted_iota(jnp.int32, (BLK, NT), 1)
    oh = (((tok == cols) & (tok != 0)) | (cid == cols)).astype(jnp.float32)
    aa_ch = lax.dot_general(oh, tab_ref[...], (((1,), (0,)), ((), ())),
                            preferred_element_type=jnp.float32)
    out_ref[...] = aa_ch + pe_ref[...]


def kernel(tokens, charges, aa_table, charge_table):
    fidx = jnp.stack(
        [tokens.reshape(B * L),
         NV + jnp.broadcast_to(charges[:, None], (B, L)).reshape(B * L)],
        axis=1)
    tab = jnp.concatenate([aa_table, charge_table], axis=0)
    out2 = pl.pallas_call(
        _body,
        grid=(B * L // BLK,),
        in_specs=[
            pl.BlockSpec((BLK, 2), lambda i: (i, 0)),
            pl.BlockSpec((NT, D), lambda i: (0, 0)),
        ],
        out_specs=pl.BlockSpec((BLK, D), lambda i: (i, 0)),
        out_shape=jax.ShapeDtypeStruct((B * L, D), jnp.float32),
        scratch_shapes=[pltpu.VMEM((BLK, D), jnp.float32)],
    )(fidx, tab)
    return out2.reshape(B, L, D)
